# baseline (speedup 1.0000x reference)
import jax
import jax.numpy as jnp
from jax import lax
from jax.experimental import pallas as pl
from jax.experimental.pallas import tpu as pltpu
from jax.experimental.pallas import tpu_sc as plsc

_NCLS = 11
_BC = 128
_LANES = 16


def _sc_body(obj_vmem, out_vmem):
    w = obj_vmem.shape[2]

    @pl.loop(0, w)
    def _w_loop(wi):
        @pl.loop(0, _BC // _LANES)
        def _v_loop(j):
            sl = pl.ds(j * _LANES, _LANES)
            vec = obj_vmem.at[0, 0, wi, sl][...]
            for c in range(_NCLS):
                out_vmem.at[c, 0, wi, sl][...] = jnp.where(
                    vec == c, jnp.float32(1.0), jnp.float32(0.0)
                )


def _sc_onehot(xt, out_shape):
    h, _c, w, b = xt.shape
    mesh = plsc.VectorSubcoreMesh(core_axis_name="core", subcore_axis_name="subcore")

    @pl.kernel(out_type=jax.ShapeDtypeStruct(out_shape, jnp.float32), mesh=mesh)
    def k(x_hbm, o_hbm):
        pltpu.emit_pipeline(
            _sc_body,
            grid=(b // _BC, h),
            in_specs=[
                pl.BlockSpec((1, 1, w, _BC), index_map=lambda i, j: (j, 0, 0, i))
            ],
            out_specs=[
                pl.BlockSpec((_NCLS, 1, w, _BC), index_map=lambda i, j: (0, j, 0, i))
            ],
            core_axis_name=("core", "subcore"),
            dimension_semantics=(pltpu.PARALLEL, pltpu.PARALLEL),
        )(x_hbm, o_hbm)

    return k(xt)


def kernel(x):
    b, h, w, _c = x.shape
    xt = jnp.transpose(x, (1, 3, 2, 0))
    out_t = _sc_onehot(xt, (_NCLS, h, w, b))
    return jnp.transpose(out_t, (3, 0, 1, 2))


# speedup vs baseline: 2.0861x; 2.0861x over previous
import jax
import jax.numpy as jnp
from jax.experimental import pallas as pl

_NCLS = 11


def _onehot_kernel(obj_ref, out_ref):
    obj = obj_ref[:, 0]
    _one, w, b = obj.shape
    cls = jax.lax.broadcasted_iota(jnp.int32, (_NCLS, 1, w, b), 0)
    out_ref[...] = (obj[None] == cls).astype(jnp.float32)


def kernel(x):
    b, h, w, _c = x.shape
    xt = jnp.transpose(x, (1, 3, 2, 0))
    out_t = pl.pallas_call(
        _onehot_kernel,
        grid=(h,),
        in_specs=[pl.BlockSpec((1, 1, w, b), lambda i: (i, 0, 0, 0))],
        out_specs=pl.BlockSpec((_NCLS, 1, w, b), lambda i: (0, i, 0, 0)),
        out_shape=jax.ShapeDtypeStruct((_NCLS, h, w, b), jnp.float32),
    )(xt)
    return jnp.transpose(out_t, (3, 0, 1, 2))


# speedup vs baseline: 2.4687x; 1.1834x over previous
import jax
import jax.numpy as jnp
from jax.experimental import pallas as pl
from jax.experimental.pallas import tpu as pltpu

_NCLS = 11
_BB = 512


def _onehot_kernel(obj_ref, out_ref):
    obj = obj_ref[:, 0]
    h, w, bb = obj.shape
    cls = jax.lax.broadcasted_iota(jnp.int32, (_NCLS, h, w, bb), 0)
    out_ref[...] = (obj[None] == cls).astype(jnp.float32)


def kernel(x):
    b, h, w, _c = x.shape
    xt = jnp.transpose(x, (1, 3, 2, 0))
    out_t = pl.pallas_call(
        _onehot_kernel,
        grid=(b // _BB,),
        in_specs=[pl.BlockSpec((h, 1, w, _BB), lambda i: (0, 0, 0, i))],
        out_specs=pl.BlockSpec((_NCLS, h, w, _BB), lambda i: (0, 0, 0, i)),
        out_shape=jax.ShapeDtypeStruct((_NCLS, h, w, b), jnp.float32),
        compiler_params=pltpu.CompilerParams(dimension_semantics=("parallel",)),
    )(xt)
    return jnp.transpose(out_t, (3, 0, 1, 2))
